# baseline (speedup 1.0000x reference)
import numpy as np
import jax
import jax.numpy as jnp
from jax import lax
from jax.experimental import pallas as pl
from jax.experimental.pallas import tpu as pltpu
from jax.experimental.pallas import tpu_sc as plsc

_B = 16384
_CAT_SLICES = [(1, 10), (12, 29), (30, 33), (33, 40), (40, 64), (64, 79),
               (79, 84), (84, 94), (94, 96), (96, 99), (99, 105), (105, 113),
               (116, 122), (122, 128), (128, 151), (151, 159), (160, 165)]
_CONT_COLS = [0, 10, 11, 29, 113, 114, 115, 159, 165, 166, 167]
_NBINS = 800
_EPS = 1e-10
_NB = 4
_R = _B // _NB

_CHUNK = 5376
_CHUNK2 = 5632
_NV = _CHUNK // 128
_NV2 = _CHUNK2 // 128
_ESTRIDE = 808


def _seg_matrix():
    s = np.zeros((168, len(_CAT_SLICES)), dtype=np.float32)
    for j, (a, b) in enumerate(_CAT_SLICES):
        s[a:b, j] = 1.0
    return s


def _cont_mask():
    m = np.zeros((1, 168), dtype=np.float32)
    m[0, _CONT_COLS] = 1.0
    return m


def _cat_mask():
    m = np.zeros((1, 168), dtype=np.float32)
    for (a, b) in _CAT_SLICES:
        m[0, a:b] = 1.0
    return m


def _minmax_kernel(enc_ref, mm_ref, et_ref):
    enc = enc_ref[...]
    mm_ref[0:1, 0:10] = jnp.min(enc, axis=0, keepdims=True)
    mm_ref[1:2, 0:10] = jnp.max(enc, axis=0, keepdims=True)
    et_ref[...] = enc.T


def _hist_kernel(enc_hbm, lab_hbm, edges_hbm, out_hbm,
                 data_v, lab_v, edges_v, hist_v, row_v):
    wid = lax.axis_index("c") * 16 + lax.axis_index("s")
    chunk = wid // 10
    col = wid % 10
    start = jnp.minimum(chunk, 2) * _CHUNK
    n_out = jnp.where(wid >= 30, 0, jnp.where(chunk == 2, _NV2, _NV))

    pltpu.sync_copy(enc_hbm.at[pl.ds(col * _B + start, _CHUNK2)], data_v)
    pltpu.sync_copy(lab_hbm.at[pl.ds(start, _CHUNK2)], lab_v)
    pltpu.sync_copy(edges_hbm, edges_v)

    ebase = jnp.full((16,), col * _ESTRIDE, jnp.int32)
    mn = plsc.load_gather(edges_v, [ebase])
    mx = plsc.load_gather(edges_v, [ebase + _NBINS])
    invw = jnp.float32(_NBINS) / (mx - mn)
    lane = lax.broadcasted_iota(jnp.int32, (16,), 0)
    ones_f = jnp.ones((16,), jnp.float32)

    def _zero(i, _):
        for k in range(16):
            hist_v[pl.ds(i * 256 + k * 16, 16)] = jnp.zeros((16,), jnp.float32)
        return 0
    lax.fori_loop(0, 100, _zero, 0)

    def _bin_rows(o, _):
        for k in range(8):
            i = o * 8 + k
            v = data_v[pl.ds(i * 16, 16)]
            g = lab_v[pl.ds(i * 16, 16)]
            b0 = jnp.clip(((v - mn) * invw).astype(jnp.int32), 0, _NBINS - 1)
            e0 = plsc.load_gather(edges_v, [ebase + b0])
            e1 = plsc.load_gather(edges_v, [ebase + b0 + 1])
            b = b0 + (v >= e1).astype(jnp.int32) - (v < e0).astype(jnp.int32)
            b = jnp.clip(b, 0, _NBINS - 1)
            idx = lane * 1600 + g.astype(jnp.int32) * 800 + b
            plsc.addupdate_scatter(hist_v, [idx], ones_f)
        return 0
    lax.fori_loop(0, n_out, _bin_rows, 0)

    def _reduce(jj, _):
        a0 = hist_v[pl.ds(jj * 32, 16)]
        a1 = hist_v[pl.ds(jj * 32 + 16, 16)]
        for l in range(1, 16):
            a0 = a0 + hist_v[pl.ds(l * 1600 + jj * 32, 16)]
            a1 = a1 + hist_v[pl.ds(l * 1600 + jj * 32 + 16, 16)]
        row_v[pl.ds(jj * 32, 16)] = a0
        row_v[pl.ds(jj * 32 + 16, 16)] = a1
        return 0
    lax.fori_loop(0, 50, _reduce, 0)

    pltpu.sync_copy(row_v, out_hbm.at[wid])


def _stats_kernel(dec_ref, true_ref, lab_ref, seg_ref, cmask_ref,
                  kmask_ref, parts_ref, out_ref, smem_acc):
    j = pl.program_id(0)

    @pl.when(j == 0)
    def _init():
        smem_acc[0] = 0.0
        smem_acc[1] = 0.0
        smem_acc[2] = 0.0

    dec = dec_ref[...]
    true = true_ref[...]
    diff = dec - true
    mse_part = jnp.sum(diff * diff * cmask_ref[...])

    e = jnp.exp(dec)
    sumexp = lax.dot_general(e, seg_ref[...], (((1,), (0,)), ((), ())),
                             preferred_element_type=jnp.float32)
    ce_part = jnp.sum(jnp.log(sumexp)) - jnp.sum(dec * true * kmask_ref[...])

    smem_acc[0] += mse_part
    smem_acc[1] += ce_part
    smem_acc[2] += jnp.sum(lab_ref[...])

    @pl.when(j == _NB - 1)
    def _final():
        n_f = smem_acc[2]
        n_m = jnp.float32(_B) - n_f
        kl = jnp.float32(0.0)
        for c in range(10):
            a = (parts_ref[c:c + 1, :] + parts_ref[c + 10:c + 11, :]
                 + parts_ref[c + 20:c + 21, :])
            pp = a[:, :_NBINS] / n_m
            qq = a[:, _NBINS:] / n_f
            mm = 0.5 * (pp + qq)
            kl += jnp.sum(pp * jnp.log((pp + _EPS) / (mm + _EPS)))
            kl += jnp.sum(qq * jnp.log((qq + _EPS) / (mm + _EPS)))
        kld = 0.5 * kl

        inv_b = jnp.float32(1.0 / _B)
        mse_loss = smem_acc[0] * inv_b
        ce_loss = smem_acc[1] * inv_b
        ajsd = 0.5 * kld
        multi = 0.5 * (mse_loss + ce_loss) + ajsd

        lane = lax.broadcasted_iota(jnp.int32, (1, 128), 1)
        out_ref[...] = jnp.where(lane == 0, multi,
                       jnp.where(lane == 1, mse_loss,
                       jnp.where(lane == 2, ce_loss, ajsd)))


def kernel(data_encoded, data_decoded, data_true, label_true, batch_size):
    seg = jnp.asarray(_seg_matrix())
    cmask = jnp.asarray(_cont_mask())
    kmask = jnp.asarray(_cat_mask())
    lab_flat = label_true[:, 1]
    lab_row = lab_flat.reshape(1, _B)

    info, enc_t = pl.pallas_call(
        _minmax_kernel,
        out_shape=[
            jax.ShapeDtypeStruct((2, 128), jnp.float32),
            jax.ShapeDtypeStruct((10, _B), jnp.float32),
        ],
    )(data_encoded)
    enc_t_flat = enc_t.reshape(-1)
    edges = [jnp.linspace(info[0, c], info[1, c], _NBINS + 1)
             for c in range(10)]
    etab = jnp.stack([jnp.pad(e, (0, _ESTRIDE - _NBINS - 1)) for e in edges])
    etab = etab.reshape(-1)

    mesh = plsc.VectorSubcoreMesh(core_axis_name="c", subcore_axis_name="s")
    parts = pl.kernel(
        _hist_kernel,
        mesh=mesh,
        out_type=jax.ShapeDtypeStruct((32, 1600), jnp.float32),
        scratch_types=[
            pltpu.VMEM((_CHUNK2,), jnp.float32),
            pltpu.VMEM((_CHUNK2,), jnp.float32),
            pltpu.VMEM((10 * _ESTRIDE,), jnp.float32),
            pltpu.VMEM((25600,), jnp.float32),
            pltpu.VMEM((1600,), jnp.float32),
        ],
        compiler_params=pltpu.CompilerParams(needs_layout_passes=False),
    )(enc_t_flat, lab_flat, etab)

    out = pl.pallas_call(
        _stats_kernel,
        grid=(_NB,),
        in_specs=[
            pl.BlockSpec((_R, 168), lambda j: (j, 0)),
            pl.BlockSpec((_R, 168), lambda j: (j, 0)),
            pl.BlockSpec((1, _R), lambda j: (0, j)),
            pl.BlockSpec((168, 17), lambda j: (0, 0)),
            pl.BlockSpec((1, 168), lambda j: (0, 0)),
            pl.BlockSpec((1, 168), lambda j: (0, 0)),
            pl.BlockSpec((32, 1600), lambda j: (0, 0)),
        ],
        out_specs=pl.BlockSpec((1, 128), lambda j: (0, 0)),
        out_shape=jax.ShapeDtypeStruct((1, 128), jnp.float32),
        scratch_shapes=[pltpu.SMEM((4,), jnp.float32)],
        compiler_params=pltpu.CompilerParams(
            dimension_semantics=("arbitrary",),
        ),
    )(data_decoded, data_true, lab_row, seg, cmask, kmask, parts)
    return (out[0, 0], out[0, 1], out[0, 2], out[0, 3])


# speedup vs baseline: 1.0229x; 1.0229x over previous
import numpy as np
import jax
import jax.numpy as jnp
from jax import lax
from jax.experimental import pallas as pl
from jax.experimental.pallas import tpu as pltpu
from jax.experimental.pallas import tpu_sc as plsc

_B = 16384
_CAT_SLICES = [(1, 10), (12, 29), (30, 33), (33, 40), (40, 64), (64, 79),
               (79, 84), (84, 94), (94, 96), (96, 99), (99, 105), (105, 113),
               (116, 122), (122, 128), (128, 151), (151, 159), (160, 165)]
_CONT_COLS = [0, 10, 11, 29, 113, 114, 115, 159, 165, 166, 167]
_NBINS = 800
_EPS = 1e-10
_NB = 4
_R = _B // _NB

_CHUNK = 5376
_CHUNK2 = 5632
_NV = _CHUNK // 128
_NV2 = _CHUNK2 // 128
_ESTRIDE = 808


def _seg_matrix():
    s = np.zeros((168, len(_CAT_SLICES)), dtype=np.float32)
    for j, (a, b) in enumerate(_CAT_SLICES):
        s[a:b, j] = 1.0
    return s


def _cont_mask():
    m = np.zeros((1, 168), dtype=np.float32)
    m[0, _CONT_COLS] = 1.0
    return m


def _cat_mask():
    m = np.zeros((1, 168), dtype=np.float32)
    for (a, b) in _CAT_SLICES:
        m[0, a:b] = 1.0
    return m


def _minmax_kernel(enc_ref, mm_ref):
    enc = enc_ref[...]
    mm_ref[0:1, 0:10] = jnp.min(enc, axis=0, keepdims=True)
    mm_ref[1:2, 0:10] = jnp.max(enc, axis=0, keepdims=True)


def _hist_kernel(enc_hbm, lab_hbm, edges_hbm, out_hbm,
                 data_v, lab_v, edges_v, hist_v, row_v):
    wid = lax.axis_index("c") * 16 + lax.axis_index("s")
    chunk = wid // 10
    col = wid % 10
    start = jnp.minimum(chunk, 2) * _CHUNK
    n_out = jnp.where(wid >= 30, 0, jnp.where(chunk == 2, _NV2, _NV))

    pltpu.sync_copy(enc_hbm.at[pl.ds(col * _B + start, _CHUNK2)], data_v)
    pltpu.sync_copy(lab_hbm.at[pl.ds(start, _CHUNK2)], lab_v)
    pltpu.sync_copy(edges_hbm, edges_v)

    ebase = jnp.full((16,), col * _ESTRIDE, jnp.int32)
    mn = plsc.load_gather(edges_v, [ebase])
    mx = plsc.load_gather(edges_v, [ebase + _NBINS])
    invw = jnp.float32(_NBINS) / (mx - mn)
    lane = lax.broadcasted_iota(jnp.int32, (16,), 0)
    ones_f = jnp.ones((16,), jnp.float32)

    def _zero(i, _):
        for k in range(16):
            hist_v[pl.ds(i * 256 + k * 16, 16)] = jnp.zeros((16,), jnp.float32)
        return 0
    lax.fori_loop(0, 100, _zero, 0)

    def _bin_rows(o, _):
        for k in range(8):
            i = o * 8 + k
            v = data_v[pl.ds(i * 16, 16)]
            g = lab_v[pl.ds(i * 16, 16)]
            b0 = jnp.clip(((v - mn) * invw).astype(jnp.int32), 0, _NBINS - 1)
            e0 = plsc.load_gather(edges_v, [ebase + b0])
            e1 = plsc.load_gather(edges_v, [ebase + b0 + 1])
            b = b0 + (v >= e1).astype(jnp.int32) - (v < e0).astype(jnp.int32)
            b = jnp.clip(b, 0, _NBINS - 1)
            idx = lane * 1600 + g.astype(jnp.int32) * 800 + b
            plsc.addupdate_scatter(hist_v, [idx], ones_f)
        return 0
    lax.fori_loop(0, n_out, _bin_rows, 0)

    def _reduce(jj, _):
        a0 = hist_v[pl.ds(jj * 32, 16)]
        a1 = hist_v[pl.ds(jj * 32 + 16, 16)]
        for l in range(1, 16):
            a0 = a0 + hist_v[pl.ds(l * 1600 + jj * 32, 16)]
            a1 = a1 + hist_v[pl.ds(l * 1600 + jj * 32 + 16, 16)]
        row_v[pl.ds(jj * 32, 16)] = a0
        row_v[pl.ds(jj * 32 + 16, 16)] = a1
        return 0
    lax.fori_loop(0, 50, _reduce, 0)

    pltpu.sync_copy(row_v, out_hbm.at[wid])


def _stats_kernel(dec_ref, true_ref, lab_ref, seg_ref, cmask_ref,
                  kmask_ref, parts_ref, out_ref, smem_acc):
    j = pl.program_id(0)

    @pl.when(j == 0)
    def _init():
        smem_acc[0] = 0.0
        smem_acc[1] = 0.0
        smem_acc[2] = 0.0

    dec = dec_ref[...]
    true = true_ref[...]
    diff = dec - true
    mse_part = jnp.sum(diff * diff * cmask_ref[...])

    e = jnp.exp(dec)
    sumexp = lax.dot_general(e, seg_ref[...], (((1,), (0,)), ((), ())),
                             preferred_element_type=jnp.float32)
    ce_part = jnp.sum(jnp.log(sumexp)) - jnp.sum(dec * true * kmask_ref[...])

    smem_acc[0] += mse_part
    smem_acc[1] += ce_part
    smem_acc[2] += jnp.sum(lab_ref[...])

    @pl.when(j == _NB - 1)
    def _final():
        n_f = smem_acc[2]
        n_m = jnp.float32(_B) - n_f
        kl = jnp.float32(0.0)
        for c in range(10):
            a = (parts_ref[c:c + 1, :] + parts_ref[c + 10:c + 11, :]
                 + parts_ref[c + 20:c + 21, :])
            pp = a[:, :_NBINS] / n_m
            qq = a[:, _NBINS:] / n_f
            mm = 0.5 * (pp + qq)
            kl += jnp.sum(pp * jnp.log((pp + _EPS) / (mm + _EPS)))
            kl += jnp.sum(qq * jnp.log((qq + _EPS) / (mm + _EPS)))
        kld = 0.5 * kl

        inv_b = jnp.float32(1.0 / _B)
        mse_loss = smem_acc[0] * inv_b
        ce_loss = smem_acc[1] * inv_b
        ajsd = 0.5 * kld
        multi = 0.5 * (mse_loss + ce_loss) + ajsd

        lane = lax.broadcasted_iota(jnp.int32, (1, 128), 1)
        out_ref[...] = jnp.where(lane == 0, multi,
                       jnp.where(lane == 1, mse_loss,
                       jnp.where(lane == 2, ce_loss, ajsd)))


def kernel(data_encoded, data_decoded, data_true, label_true, batch_size):
    seg = jnp.asarray(_seg_matrix())
    cmask = jnp.asarray(_cont_mask())
    kmask = jnp.asarray(_cat_mask())
    lab_flat = label_true[:, 1]
    lab_row = lab_flat.reshape(1, _B)
    enc_t_flat = data_encoded.T.reshape(-1)

    info = pl.pallas_call(
        _minmax_kernel,
        out_shape=jax.ShapeDtypeStruct((2, 128), jnp.float32),
    )(data_encoded)
    edges = [jnp.linspace(info[0, c], info[1, c], _NBINS + 1)
             for c in range(10)]
    etab = jnp.stack([jnp.pad(e, (0, _ESTRIDE - _NBINS - 1)) for e in edges])
    etab = etab.reshape(-1)

    mesh = plsc.VectorSubcoreMesh(core_axis_name="c", subcore_axis_name="s")
    parts = pl.kernel(
        _hist_kernel,
        mesh=mesh,
        out_type=jax.ShapeDtypeStruct((32, 1600), jnp.float32),
        scratch_types=[
            pltpu.VMEM((_CHUNK2,), jnp.float32),
            pltpu.VMEM((_CHUNK2,), jnp.float32),
            pltpu.VMEM((10 * _ESTRIDE,), jnp.float32),
            pltpu.VMEM((25600,), jnp.float32),
            pltpu.VMEM((1600,), jnp.float32),
        ],
        compiler_params=pltpu.CompilerParams(needs_layout_passes=False),
    )(enc_t_flat, lab_flat, etab)

    out = pl.pallas_call(
        _stats_kernel,
        grid=(_NB,),
        in_specs=[
            pl.BlockSpec((_R, 168), lambda j: (j, 0)),
            pl.BlockSpec((_R, 168), lambda j: (j, 0)),
            pl.BlockSpec((1, _R), lambda j: (0, j)),
            pl.BlockSpec((168, 17), lambda j: (0, 0)),
            pl.BlockSpec((1, 168), lambda j: (0, 0)),
            pl.BlockSpec((1, 168), lambda j: (0, 0)),
            pl.BlockSpec((32, 1600), lambda j: (0, 0)),
        ],
        out_specs=pl.BlockSpec((1, 128), lambda j: (0, 0)),
        out_shape=jax.ShapeDtypeStruct((1, 128), jnp.float32),
        scratch_shapes=[pltpu.SMEM((4,), jnp.float32)],
        compiler_params=pltpu.CompilerParams(
            dimension_semantics=("arbitrary",),
        ),
    )(data_decoded, data_true, lab_row, seg, cmask, kmask, parts)
    return (out[0, 0], out[0, 1], out[0, 2], out[0, 3])


# speedup vs baseline: 1.0401x; 1.0167x over previous
import numpy as np
import jax
import jax.numpy as jnp
from jax import lax
from jax.experimental import pallas as pl
from jax.experimental.pallas import tpu as pltpu
from jax.experimental.pallas import tpu_sc as plsc

_B = 16384
_CAT_SLICES = [(1, 10), (12, 29), (30, 33), (33, 40), (40, 64), (64, 79),
               (79, 84), (84, 94), (94, 96), (96, 99), (99, 105), (105, 113),
               (116, 122), (122, 128), (128, 151), (151, 159), (160, 165)]
_CONT_COLS = [0, 10, 11, 29, 113, 114, 115, 159, 165, 166, 167]
_NBINS = 800
_EPS = 1e-10
_NB = 4
_R = _B // _NB

_CHUNK = 5376
_CHUNK2 = 5632
_NV = _CHUNK // 128
_NV2 = _CHUNK2 // 128
_ESTRIDE = 801


def _seg_matrix():
    s = np.zeros((168, len(_CAT_SLICES)), dtype=np.float32)
    for j, (a, b) in enumerate(_CAT_SLICES):
        s[a:b, j] = 1.0
    return s


def _cont_mask():
    m = np.zeros((1, 168), dtype=np.float32)
    m[0, _CONT_COLS] = 1.0
    return m


def _cat_mask():
    m = np.zeros((1, 168), dtype=np.float32)
    for (a, b) in _CAT_SLICES:
        m[0, a:b] = 1.0
    return m


def _minmax_kernel(enc_ref, mm_ref):
    enc = enc_ref[...]
    mm_ref[0:1, 0:10] = jnp.min(enc, axis=0, keepdims=True)
    mm_ref[1:2, 0:10] = jnp.max(enc, axis=0, keepdims=True)


def _hist_kernel(enc_hbm, lab_hbm, edges_hbm, out_hbm,
                 data_v, lab_v, edges_v, hist_v, row_v):
    wid = lax.axis_index("c") * 16 + lax.axis_index("s")
    chunk = wid // 10
    col = wid % 10
    start = jnp.minimum(chunk, 2) * _CHUNK
    n_out = jnp.where(wid >= 30, 0, jnp.where(chunk == 2, _NV2, _NV))

    pltpu.sync_copy(enc_hbm.at[pl.ds(col * _B + start, _CHUNK2)], data_v)
    pltpu.sync_copy(lab_hbm.at[pl.ds(start, _CHUNK2)], lab_v)
    pltpu.sync_copy(edges_hbm, edges_v)

    ebase = jnp.full((16,), col * _ESTRIDE, jnp.int32)
    mn = plsc.load_gather(edges_v, [ebase])
    mx = plsc.load_gather(edges_v, [ebase + _NBINS])
    invw = jnp.float32(_NBINS) / (mx - mn)
    lane = lax.broadcasted_iota(jnp.int32, (16,), 0)
    ones_f = jnp.ones((16,), jnp.float32)

    def _zero(i, _):
        for k in range(16):
            hist_v[pl.ds(i * 256 + k * 16, 16)] = jnp.zeros((16,), jnp.float32)
        return 0
    lax.fori_loop(0, 100, _zero, 0)

    def _bin_rows(o, _):
        for k in range(8):
            i = o * 8 + k
            v = data_v[pl.ds(i * 16, 16)]
            g = lab_v[pl.ds(i * 16, 16)]
            b0 = jnp.clip(((v - mn) * invw).astype(jnp.int32), 0, _NBINS - 1)
            e0 = plsc.load_gather(edges_v, [ebase + b0])
            e1 = plsc.load_gather(edges_v, [ebase + b0 + 1])
            b = b0 + (v >= e1).astype(jnp.int32) - (v < e0).astype(jnp.int32)
            b = jnp.clip(b, 0, _NBINS - 1)
            idx = lane * 1600 + g.astype(jnp.int32) * 800 + b
            plsc.addupdate_scatter(hist_v, [idx], ones_f)
        return 0
    lax.fori_loop(0, n_out, _bin_rows, 0)

    def _reduce(jj, _):
        a0 = hist_v[pl.ds(jj * 32, 16)]
        a1 = hist_v[pl.ds(jj * 32 + 16, 16)]
        for l in range(1, 16):
            a0 = a0 + hist_v[pl.ds(l * 1600 + jj * 32, 16)]
            a1 = a1 + hist_v[pl.ds(l * 1600 + jj * 32 + 16, 16)]
        row_v[pl.ds(jj * 32, 16)] = a0
        row_v[pl.ds(jj * 32 + 16, 16)] = a1
        return 0
    lax.fori_loop(0, 50, _reduce, 0)

    pltpu.sync_copy(row_v, out_hbm.at[wid])


def _stats_kernel(dec_ref, true_ref, lab_ref, seg_ref, cmask_ref,
                  kmask_ref, parts_ref, out_ref, smem_acc):
    j = pl.program_id(0)

    @pl.when(j == 0)
    def _init():
        smem_acc[0] = 0.0
        smem_acc[1] = 0.0
        smem_acc[2] = 0.0

    dec = dec_ref[...]
    true = true_ref[...]
    diff = dec - true
    mse_part = jnp.sum(diff * diff * cmask_ref[...])

    e = jnp.exp(dec)
    sumexp = lax.dot_general(e, seg_ref[...], (((1,), (0,)), ((), ())),
                             preferred_element_type=jnp.float32)
    ce_part = jnp.sum(jnp.log(sumexp)) - jnp.sum(dec * true * kmask_ref[...])

    smem_acc[0] += mse_part
    smem_acc[1] += ce_part
    smem_acc[2] += jnp.sum(lab_ref[...])

    @pl.when(j == _NB - 1)
    def _final():
        n_f = smem_acc[2]
        n_m = jnp.float32(_B) - n_f
        kl = jnp.float32(0.0)
        for c in range(10):
            a = (parts_ref[c:c + 1, :] + parts_ref[c + 10:c + 11, :]
                 + parts_ref[c + 20:c + 21, :])
            pp = a[:, :_NBINS] / n_m
            qq = a[:, _NBINS:] / n_f
            mm = 0.5 * (pp + qq)
            kl += jnp.sum(pp * jnp.log((pp + _EPS) / (mm + _EPS)))
            kl += jnp.sum(qq * jnp.log((qq + _EPS) / (mm + _EPS)))
        kld = 0.5 * kl

        inv_b = jnp.float32(1.0 / _B)
        mse_loss = smem_acc[0] * inv_b
        ce_loss = smem_acc[1] * inv_b
        ajsd = 0.5 * kld
        multi = 0.5 * (mse_loss + ce_loss) + ajsd

        lane = lax.broadcasted_iota(jnp.int32, (1, 128), 1)
        out_ref[...] = jnp.where(lane == 0, multi,
                       jnp.where(lane == 1, mse_loss,
                       jnp.where(lane == 2, ce_loss, ajsd)))


def kernel(data_encoded, data_decoded, data_true, label_true, batch_size):
    seg = jnp.asarray(_seg_matrix())
    cmask = jnp.asarray(_cont_mask())
    kmask = jnp.asarray(_cat_mask())
    lab_flat = label_true[:, 1]
    lab_row = lab_flat.reshape(1, _B)
    enc_t_flat = data_encoded.T.reshape(-1)

    info = pl.pallas_call(
        _minmax_kernel,
        out_shape=jax.ShapeDtypeStruct((2, 128), jnp.float32),
    )(data_encoded)
    edges = [jnp.linspace(info[0, c], info[1, c], _NBINS + 1)
             for c in range(10)]
    etab = jnp.concatenate(edges)

    mesh = plsc.VectorSubcoreMesh(core_axis_name="c", subcore_axis_name="s")
    parts = pl.kernel(
        _hist_kernel,
        mesh=mesh,
        out_type=jax.ShapeDtypeStruct((32, 1600), jnp.float32),
        scratch_types=[
            pltpu.VMEM((_CHUNK2,), jnp.float32),
            pltpu.VMEM((_CHUNK2,), jnp.float32),
            pltpu.VMEM((10 * _ESTRIDE,), jnp.float32),
            pltpu.VMEM((25600,), jnp.float32),
            pltpu.VMEM((1600,), jnp.float32),
        ],
        compiler_params=pltpu.CompilerParams(needs_layout_passes=False),
    )(enc_t_flat, lab_flat, etab)

    out = pl.pallas_call(
        _stats_kernel,
        grid=(_NB,),
        in_specs=[
            pl.BlockSpec((_R, 168), lambda j: (j, 0)),
            pl.BlockSpec((_R, 168), lambda j: (j, 0)),
            pl.BlockSpec((1, _R), lambda j: (0, j)),
            pl.BlockSpec((168, 17), lambda j: (0, 0)),
            pl.BlockSpec((1, 168), lambda j: (0, 0)),
            pl.BlockSpec((1, 168), lambda j: (0, 0)),
            pl.BlockSpec((32, 1600), lambda j: (0, 0)),
        ],
        out_specs=pl.BlockSpec((1, 128), lambda j: (0, 0)),
        out_shape=jax.ShapeDtypeStruct((1, 128), jnp.float32),
        scratch_shapes=[pltpu.SMEM((4,), jnp.float32)],
        compiler_params=pltpu.CompilerParams(
            dimension_semantics=("arbitrary",),
        ),
    )(data_decoded, data_true, lab_row, seg, cmask, kmask, parts)
    return (out[0, 0], out[0, 1], out[0, 2], out[0, 3])
